# Initial kernel scaffold; baseline (speedup 1.0000x reference)
import jax
import jax.numpy as jnp
from jax.experimental import pallas as pl


def kernel(local_cellxregion_ix, regions_oi, cells_oi, baseline_weight, lib):
    raise NotImplementedError("write your pallas kernel here")


# speedup vs baseline: 24.7466x; 24.7466x over previous
import functools

import jax
import jax.numpy as jnp
from jax import lax
from jax.experimental import pallas as pl
from jax.experimental.pallas import tpu as pltpu
from jax.experimental.pallas import tpu_sc as plsc

NF = 8388608
N_CELLS = 4096
N_REGIONS = 1024
NB = N_CELLS * N_REGIONS

NC = 2
NS = 16

NPASS = 2
Q = NB // (NC * NPASS)
DUMMY = 2048
FPT = NF // NS
C = 4096
K = FPT // C
ROWS = C // 128
QP = Q // NS


def _sc_body(frag, regions2, cells2, baseline, lib,
             count_out, rb_out, cl_out,
             hist, idxbuf, cbuf, zbuf, ones,
             sem_in, sem_s0, sem_s1, sem_g):
    c = lax.axis_index("c")
    s = lax.axis_index("s")

    zv = jnp.zeros((16,), jnp.float32)
    ov = jnp.ones((16,), jnp.float32)
    iv = lax.iota(jnp.int32, 16)

    def _z16(i, _):
        zbuf[pl.ds(i * 16, 16)] = zv
        return 0
    lax.fori_loop(0, C // 16, _z16, 0)
    for jj in range(8):
        ones[pl.ds(jj * 16, 16)] = ov

    dvec0 = (iv * 128 + s * 8) & (DUMMY - 1)

    for p in range(NPASS):
        r = NPASS * p + c
        base = r * Q

        hoff = s * QP

        def _zh(i, _):
            pltpu.sync_copy(zbuf, hist.at[pl.ds(hoff + i * C, C)])
            return 0
        lax.fori_loop(0, QP // C, _zh, 0)
        plsc.subcore_barrier()

        pltpu.async_copy(frag.at[pl.ds(s * FPT, C)], idxbuf.at[0], sem_in)

        def _chunk(k, dvec):
            kb = lax.rem(k, 2)
            pltpu.make_async_copy(frag.at[pl.ds(0, C)], idxbuf.at[0],
                                  sem_in).wait()

            @pl.when(k + 1 < K)
            def _():
                pltpu.async_copy(frag.at[pl.ds(s * FPT + (k + 1) * C, C)],
                                 idxbuf.at[1 - kb], sem_in)

            @pl.when(jnp.logical_and(k >= 2, kb == 0))
            def _():
                pltpu.make_async_copy(hist.at[pl.ds(0, C)], zbuf,
                                      sem_s0).wait()

            @pl.when(jnp.logical_and(k >= 2, kb == 1))
            def _():
                pltpu.make_async_copy(hist.at[pl.ds(0, C)], zbuf,
                                      sem_s1).wait()

            def _row(j, dv):
                for jj in range(8):
                    v = idxbuf[kb, pl.ds(j * 128 + jj * 16, 16)]
                    loc = v - base
                    m = jnp.logical_and(loc >= 0, loc < Q)
                    dv = (dv + 61) & (DUMMY - 1)
                    cbuf[kb, j, pl.ds(jj * 16, 16)] = jnp.where(m, loc, Q + dv)
                return dv
            dvec = lax.fori_loop(0, ROWS, _row, dvec)

            def _fire0(j, _):
                pltpu.async_copy(ones, hist.at[cbuf.at[kb, j]], sem_s0,
                                 add=True)
                return 0

            def _fire1(j, _):
                pltpu.async_copy(ones, hist.at[cbuf.at[kb, j]], sem_s1,
                                 add=True)
                return 0

            @pl.when(kb == 0)
            def _():
                lax.fori_loop(0, ROWS, _fire0, 0)

            @pl.when(kb == 1)
            def _():
                lax.fori_loop(0, ROWS, _fire1, 0)
            return dvec

        lax.fori_loop(0, K, _chunk, dvec0)

        pltpu.make_async_copy(hist.at[pl.ds(0, C)], zbuf, sem_s0).wait()
        pltpu.make_async_copy(hist.at[pl.ds(0, C)], zbuf, sem_s1).wait()
        plsc.subcore_barrier()
        pltpu.sync_copy(hist.at[pl.ds(s * QP, QP)],
                        count_out.at[pl.ds(base + s * QP, QP)])

    @pl.when(jnp.logical_and(c == 0, s == 1))
    def _():
        pltpu.sync_copy(regions2, cbuf.at[0, pl.ds(0, 8)])
        for j in range(8):
            pltpu.async_copy(baseline.at[cbuf.at[0, j]],
                             zbuf.at[pl.ds(j * 128, 128)], sem_g).wait()
        pltpu.sync_copy(zbuf.at[pl.ds(0, 1024)], rb_out)

    @pl.when(jnp.logical_and(c == 1, s == 1))
    def _():
        pltpu.sync_copy(cells2, cbuf.at[0])
        for j in range(32):
            pltpu.async_copy(lib.at[cbuf.at[0, j]],
                             zbuf.at[pl.ds(j * 128, 128)], sem_g).wait()
        pltpu.sync_copy(zbuf, cl_out)


@functools.cache
def _sc_histogram():
    return functools.partial(
        pl.kernel,
        out_type=(
            jax.ShapeDtypeStruct((NB,), jnp.float32),
            jax.ShapeDtypeStruct((N_REGIONS,), jnp.float32),
            jax.ShapeDtypeStruct((N_CELLS,), jnp.float32),
        ),
        mesh=plsc.VectorSubcoreMesh(core_axis_name="c", subcore_axis_name="s",
                                    num_cores=NC, num_subcores=NS),
        compiler_params=pltpu.CompilerParams(needs_layout_passes=False),
        scratch_types=[
            pltpu.VMEM_SHARED((Q + DUMMY,), jnp.float32),
            pltpu.VMEM((2, C), jnp.int32),
            pltpu.VMEM((2, ROWS, 128), jnp.int32),
            pltpu.VMEM((C,), jnp.float32),
            pltpu.VMEM((128,), jnp.float32),
            pltpu.SemaphoreType.DMA,
            pltpu.SemaphoreType.DMA,
            pltpu.SemaphoreType.DMA,
            pltpu.SemaphoreType.DMA,
        ],
    )(_sc_body)


BLK = 512


def _gammaln1p(c):
    x = c + 1.0
    p = (x * (x + 1.0) * (x + 2.0) * (x + 3.0)
         * (x + 4.0) * (x + 5.0) * (x + 6.0) * (x + 7.0))
    z = x + 8.0
    zi = 1.0 / z
    zi2 = zi * zi
    series = zi * (1.0 / 12.0 - zi2 * (1.0 / 360.0 - zi2 * (1.0 / 1260.0)))
    lg = (z - 0.5) * jnp.log(z) - z + 0.9189385332046727 + series
    return lg - jnp.log(p)


def _tc_body(count_ref, cl_ref, rb_ref, out_ref):
    cnt = count_ref[...]
    logits = cl_ref[...] + rb_ref[...]
    rate = jnp.exp(logits)
    out_ref[...] = cnt * logits - rate - _gammaln1p(cnt)


_tc_likelihood = pl.pallas_call(
    _tc_body,
    grid=(N_CELLS // BLK,),
    in_specs=[
        pl.BlockSpec((BLK, N_REGIONS), lambda i: (i, 0)),
        pl.BlockSpec((BLK, 1), lambda i: (i, 0)),
        pl.BlockSpec((1, N_REGIONS), lambda i: (0, 0)),
    ],
    out_specs=pl.BlockSpec((BLK, N_REGIONS), lambda i: (i, 0)),
    out_shape=jax.ShapeDtypeStruct((N_CELLS, N_REGIONS), jnp.float32),
)


def kernel(local_cellxregion_ix, regions_oi, cells_oi, baseline_weight, lib):
    regions2 = regions_oi.reshape(8, 128)
    cells2 = cells_oi.reshape(32, 128)
    baseline_flat = baseline_weight.reshape(-1)
    count, rb, cl = _sc_histogram()(local_cellxregion_ix, regions2, cells2,
                                    baseline_flat, lib)
    count2 = count.reshape(N_CELLS, N_REGIONS)
    cl2 = cl.reshape(N_CELLS, 1)
    rb2 = rb.reshape(1, N_REGIONS)
    return _tc_likelihood(count2, cl2, rb2)


# speedup vs baseline: 24.7785x; 1.0013x over previous
import functools

import jax
import jax.numpy as jnp
from jax import lax
from jax.experimental import pallas as pl
from jax.experimental.pallas import tpu as pltpu
from jax.experimental.pallas import tpu_sc as plsc

NF = 8388608
N_CELLS = 4096
N_REGIONS = 1024
NB = N_CELLS * N_REGIONS

NC = 2
NS = 16

NPASS = 2
Q = NB // (NC * NPASS)
DUMMY = 65536
FPT = NF // NS
C = 4096
K = FPT // C
ROWS = C // 128
QP = Q // NS


def _sc_body(frag, regions2, cells2, baseline, lib,
             count_out, rb_out, cl_out,
             hist, idxbuf, cbuf, zbuf, ones,
             sem_in, sem_s0, sem_s1, sem_g):
    c = lax.axis_index("c")
    s = lax.axis_index("s")

    zv = jnp.zeros((16,), jnp.float32)
    ov = jnp.ones((16,), jnp.float32)
    iv = lax.iota(jnp.int32, 16)

    def _z16(i, _):
        zbuf[pl.ds(i * 16, 16)] = zv
        return 0
    lax.fori_loop(0, C // 16, _z16, 0)
    for jj in range(8):
        ones[pl.ds(jj * 16, 16)] = ov

    dvec0 = (iv * 4096 + s * 256) & (DUMMY - 1)

    for p in range(NPASS):
        r = NPASS * p + c
        base = r * Q

        hoff = s * QP

        def _zh(i, _):
            pltpu.sync_copy(zbuf, hist.at[pl.ds(hoff + i * C, C)])
            return 0
        lax.fori_loop(0, QP // C, _zh, 0)
        plsc.subcore_barrier()

        pltpu.async_copy(frag.at[pl.ds(s * FPT, C)], idxbuf.at[0], sem_in)

        def _chunk(k, dvec):
            kb = lax.rem(k, 2)
            pltpu.make_async_copy(frag.at[pl.ds(0, C)], idxbuf.at[0],
                                  sem_in).wait()

            @pl.when(k + 1 < K)
            def _():
                pltpu.async_copy(frag.at[pl.ds(s * FPT + (k + 1) * C, C)],
                                 idxbuf.at[1 - kb], sem_in)

            @pl.when(jnp.logical_and(k >= 2, kb == 0))
            def _():
                pltpu.make_async_copy(hist.at[pl.ds(0, C)], zbuf,
                                      sem_s0).wait()

            @pl.when(jnp.logical_and(k >= 2, kb == 1))
            def _():
                pltpu.make_async_copy(hist.at[pl.ds(0, C)], zbuf,
                                      sem_s1).wait()

            def _row(j, dv):
                for jj in range(8):
                    v = idxbuf[kb, pl.ds(j * 128 + jj * 16, 16)]
                    loc = v - base
                    m = jnp.logical_and(loc >= 0, loc < Q)
                    dv = (dv + 61) & (DUMMY - 1)
                    cbuf[kb, j, pl.ds(jj * 16, 16)] = jnp.where(m, loc, Q + dv)
                return dv
            dvec = lax.fori_loop(0, ROWS, _row, dvec)

            def _fire0(j, _):
                pltpu.async_copy(ones, hist.at[cbuf.at[kb, j]], sem_s0,
                                 add=True)
                return 0

            def _fire1(j, _):
                pltpu.async_copy(ones, hist.at[cbuf.at[kb, j]], sem_s1,
                                 add=True)
                return 0

            @pl.when(kb == 0)
            def _():
                lax.fori_loop(0, ROWS, _fire0, 0)

            @pl.when(kb == 1)
            def _():
                lax.fori_loop(0, ROWS, _fire1, 0)
            return dvec

        lax.fori_loop(0, K, _chunk, dvec0)

        pltpu.make_async_copy(hist.at[pl.ds(0, C)], zbuf, sem_s0).wait()
        pltpu.make_async_copy(hist.at[pl.ds(0, C)], zbuf, sem_s1).wait()
        plsc.subcore_barrier()
        pltpu.sync_copy(hist.at[pl.ds(s * QP, QP)],
                        count_out.at[pl.ds(base + s * QP, QP)])

    @pl.when(jnp.logical_and(c == 0, s == 1))
    def _():
        pltpu.sync_copy(regions2, cbuf.at[0, pl.ds(0, 8)])
        for j in range(8):
            pltpu.async_copy(baseline.at[cbuf.at[0, j]],
                             zbuf.at[pl.ds(j * 128, 128)], sem_g).wait()
        pltpu.sync_copy(zbuf.at[pl.ds(0, 1024)], rb_out)

    @pl.when(jnp.logical_and(c == 1, s == 1))
    def _():
        pltpu.sync_copy(cells2, cbuf.at[0])
        for j in range(32):
            pltpu.async_copy(lib.at[cbuf.at[0, j]],
                             zbuf.at[pl.ds(j * 128, 128)], sem_g).wait()
        pltpu.sync_copy(zbuf, cl_out)


@functools.cache
def _sc_histogram():
    return functools.partial(
        pl.kernel,
        out_type=(
            jax.ShapeDtypeStruct((NB,), jnp.float32),
            jax.ShapeDtypeStruct((N_REGIONS,), jnp.float32),
            jax.ShapeDtypeStruct((N_CELLS,), jnp.float32),
        ),
        mesh=plsc.VectorSubcoreMesh(core_axis_name="c", subcore_axis_name="s",
                                    num_cores=NC, num_subcores=NS),
        compiler_params=pltpu.CompilerParams(needs_layout_passes=False),
        scratch_types=[
            pltpu.VMEM_SHARED((Q + DUMMY,), jnp.float32),
            pltpu.VMEM((2, C), jnp.int32),
            pltpu.VMEM((2, ROWS, 128), jnp.int32),
            pltpu.VMEM((C,), jnp.float32),
            pltpu.VMEM((128,), jnp.float32),
            pltpu.SemaphoreType.DMA,
            pltpu.SemaphoreType.DMA,
            pltpu.SemaphoreType.DMA,
            pltpu.SemaphoreType.DMA,
        ],
    )(_sc_body)


BLK = 512


def _gammaln1p(c):
    x = c + 1.0
    p = (x * (x + 1.0) * (x + 2.0) * (x + 3.0)
         * (x + 4.0) * (x + 5.0) * (x + 6.0) * (x + 7.0))
    z = x + 8.0
    zi = 1.0 / z
    zi2 = zi * zi
    series = zi * (1.0 / 12.0 - zi2 * (1.0 / 360.0 - zi2 * (1.0 / 1260.0)))
    lg = (z - 0.5) * jnp.log(z) - z + 0.9189385332046727 + series
    return lg - jnp.log(p)


def _tc_body(count_ref, cl_ref, rb_ref, out_ref):
    cnt = count_ref[...]
    logits = cl_ref[...] + rb_ref[...]
    rate = jnp.exp(logits)
    out_ref[...] = cnt * logits - rate - _gammaln1p(cnt)


_tc_likelihood = pl.pallas_call(
    _tc_body,
    grid=(N_CELLS // BLK,),
    in_specs=[
        pl.BlockSpec((BLK, N_REGIONS), lambda i: (i, 0)),
        pl.BlockSpec((BLK, 1), lambda i: (i, 0)),
        pl.BlockSpec((1, N_REGIONS), lambda i: (0, 0)),
    ],
    out_specs=pl.BlockSpec((BLK, N_REGIONS), lambda i: (i, 0)),
    out_shape=jax.ShapeDtypeStruct((N_CELLS, N_REGIONS), jnp.float32),
)


def kernel(local_cellxregion_ix, regions_oi, cells_oi, baseline_weight, lib):
    regions2 = regions_oi.reshape(8, 128)
    cells2 = cells_oi.reshape(32, 128)
    baseline_flat = baseline_weight.reshape(-1)
    count, rb, cl = _sc_histogram()(local_cellxregion_ix, regions2, cells2,
                                    baseline_flat, lib)
    count2 = count.reshape(N_CELLS, N_REGIONS)
    cl2 = cl.reshape(N_CELLS, 1)
    rb2 = rb.reshape(1, N_REGIONS)
    return _tc_likelihood(count2, cl2, rb2)
